# baseline (speedup 1.0000x reference)
import functools

import jax
import jax.numpy as jnp
from jax import lax
from jax.experimental import pallas as pl
from jax.experimental.pallas import tpu as pltpu
from jax.experimental.pallas import tpu_sc as plsc

_N = 10000
_E = 320000
_NT = 16
_NA = 10240
_SL = _NA // _NT
_EC = _E // _NT
_CHK = 400
_NCH = _EC // _CHK

_MESH = plsc.VectorSubcoreMesh(
    core_axis_name="c", subcore_axis_name="s", num_cores=2)


def _rsqrt16(x):
    xb = lax.bitcast_convert_type(x, jnp.int32)
    y = lax.bitcast_convert_type(jnp.int32(0x5F3759DF) - (xb >> 1), jnp.float32)
    for _ in range(3):
        y = y * (1.5 - 0.5 * x * y * y)
    return y


def _sc_norms(src_all, dst_all):
    @functools.partial(
        pl.kernel,
        out_type=jax.ShapeDtypeStruct((4, _NA), jnp.float32),
        mesh=_MESH,
        scratch_types=[
            pltpu.VMEM((_EC,), jnp.int32),
            pltpu.VMEM((_EC,), jnp.int32),
            pltpu.VMEM((_EC,), jnp.float32),
            pltpu.VMEM((_SL,), jnp.float32),
            pltpu.VMEM((_SL,), jnp.float32),
            pltpu.VMEM((_SL,), jnp.float32),
            pltpu.VMEM_SHARED((_NA,), jnp.float32),
            pltpu.VMEM_SHARED((_NA,), jnp.float32),
            pltpu.SemaphoreType.DMA,
            pltpu.SemaphoreType.DMA,
        ],
    )
    def k(src_h, dst_h, norm_out,
          sidx, didx, vals, wa, wb, zer, spA, spB, sm0, sm1):
        cid = lax.axis_index("c")
        sid = lax.axis_index("s")
        sl = pl.ds(sid * _SL, _SL)
        st0 = pltpu.async_copy(src_h.at[cid, sid], sidx, sm0)
        st1 = pltpu.async_copy(dst_h.at[cid, sid], didx, sm1)

        def fill1d(ref, n, v):
            def bb(i, _):
                ref[pl.ds(pl.multiple_of(i * 16, 16), 16)] = jnp.full(
                    (16,), v, jnp.float32)
                return 0
            lax.fori_loop(0, n // 16, bb, 0)

        fill1d(zer, _SL, 0.0)
        fill1d(vals, _EC, 1.0)
        pltpu.sync_copy(zer, spA.at[sl])
        pltpu.sync_copy(zer, spB.at[sl])
        st0.wait()
        st1.wait()
        plsc.subcore_barrier()
        h0 = pltpu.async_copy(vals, spA.at[sidx], sm0, add=True)
        h1 = pltpu.async_copy(vals, spB.at[didx], sm1, add=True)
        h0.wait()
        h1.wait()
        plsc.subcore_barrier()
        pltpu.sync_copy(spA.at[sl], wa)
        pltpu.sync_copy(spB.at[sl], wb)

        def nb(i, _):
            s = pl.ds(pl.multiple_of(i * 16, 16), 16)
            wa[s] = _rsqrt16(jnp.maximum(wa[s], 1.0))
            wb[s] = _rsqrt16(jnp.maximum(wb[s], 1.0))
            return 0
        lax.fori_loop(0, _SL // 16, nb, 0)
        pltpu.sync_copy(wa, norm_out.at[2 * cid, sl])
        pltpu.sync_copy(wb, norm_out.at[2 * cid + 1, sl])

    return k(src_all, dst_all)


def _sc_segsum(src_all, dst_all, hs, cblk=128):
    @functools.partial(
        pl.kernel,
        out_type=jax.ShapeDtypeStruct((2, _NA, cblk), jnp.float32),
        mesh=_MESH,
        scratch_types=[
            pltpu.VMEM((_CHK,), jnp.int32),
            pltpu.VMEM((_CHK,), jnp.int32),
            pltpu.VMEM((_CHK, cblk), jnp.float32),
            pltpu.VMEM_SHARED((5248, cblk), jnp.float32),
            pltpu.SemaphoreType.DMA,
            pltpu.SemaphoreType.DMA,
        ],
    )
    def k(src_h, dst_h, hs_h, out_h, sidxc, didxc, rowb, acc, sm0, sm1):
        cid = lax.axis_index("c")
        sid = lax.axis_index("s")

        hsrc = hs_h.at[cid]
        for p in (0, 1):
            def zb(i, _):
                j = i // (cblk // 16)
                c = lax.rem(i, cblk // 16)
                rowb[j, pl.ds(pl.multiple_of(c * 16, 16), 16)] = jnp.zeros(
                    (16,), jnp.float32)
                return 0
            lax.fori_loop(0, 328 * (cblk // 16), zb, 0)
            pltpu.sync_copy(rowb.at[pl.ds(0, 328)],
                            acc.at[pl.ds(sid * 328, 328)])
            plsc.subcore_barrier()

            def chunk(kk, _):
                pltpu.sync_copy(src_h.at[cid, sid, kk], sidxc)
                pltpu.sync_copy(dst_h.at[cid, sid, kk], didxc)

                def cl(i, _2):
                    s = pl.ds(pl.multiple_of(i * 16, 16), 16)
                    rel = didxc[s] - jnp.int32(p * 5120)
                    ok = (rel >= 0) & (rel < 5120)
                    didxc[s] = jnp.where(ok, rel, jnp.int32(5240))
                    return 0
                lax.fori_loop(0, _CHK // 16, cl, 0)
                pltpu.sync_copy(hsrc.at[sidxc], rowb)
                pltpu.sync_copy(rowb, acc.at[didxc], add=True)
                return 0
            lax.fori_loop(0, _NCH, chunk, 0)
            plsc.subcore_barrier()
            pltpu.sync_copy(
                acc.at[pl.ds(sid * 320, 320)],
                out_h.at[cid, pl.ds(p * 5120 + sid * 320, 320)])
            plsc.subcore_barrier()

    return k(src_all, dst_all, hs)


def _tc_prescale(x, norm4):
    cin = x.shape[1]

    def body(x_ref, n_ref, o_ref):
        xv = x_ref[...]
        zpad = jnp.zeros((_N, 128 - cin), jnp.float32)
        o_ref[0] = jnp.concatenate(
            [xv * n_ref[0, :_N].reshape(_N, 1), zpad], axis=1)
        o_ref[1] = jnp.concatenate(
            [xv * n_ref[2, :_N].reshape(_N, 1), zpad], axis=1)
    return pl.pallas_call(
        body,
        out_shape=jax.ShapeDtypeStruct((2, _N, 128), jnp.float32),
    )(x, norm4)


def _tc_layer(aggs, norm4, w0, b0, w1, b1, cout, scale_next):
    nblk = len(aggs)
    widths = [w for _, w in aggs]

    def body(*refs):
        agg_refs = refs[:nblk]
        n_ref, w0_ref, b0_ref, w1_ref, b1_ref = refs[nblk:nblk + 5]
        o_ref = refs[nblk + 5]
        s_ref = refs[nblk + 6] if scale_next else None
        nd0 = n_ref[1, :_N].reshape(_N, 1)
        nd1 = n_ref[3, :_N].reshape(_N, 1)
        h = None
        for r, (w_ref, b_ref, ndv) in enumerate(
                ((w0_ref, b0_ref, nd0), (w1_ref, b1_ref, nd1))):
            wv = w_ref[...]
            o = None
            c0 = 0
            for a_ref, cb in zip(agg_refs, widths):
                term = lax.dot_general(
                    a_ref[r][:_N, :cb] * ndv, wv[c0:c0 + cb],
                    (((1,), (0,)), ((), ())),
                    preferred_element_type=jnp.float32)
                o = term if o is None else o + term
                c0 += cb
            o = o + b_ref[...]
            h = o if h is None else h + o
        o_ref[...] = h
        if scale_next:
            npad = 128 * ((cout + 127) // 128) - cout
            hs0 = h * n_ref[0, :_N].reshape(_N, 1)
            hs1 = h * n_ref[2, :_N].reshape(_N, 1)
            if npad:
                spad = jnp.zeros((_N, npad), jnp.float32)
                hs0 = jnp.concatenate([hs0, spad], axis=1)
                hs1 = jnp.concatenate([hs1, spad], axis=1)
            s_ref[0] = hs0
            s_ref[1] = hs1

    couts = 128 * ((cout + 127) // 128)
    outs = [jax.ShapeDtypeStruct((_N, cout), jnp.float32)]
    if scale_next:
        outs.append(jax.ShapeDtypeStruct((2, _N, couts), jnp.float32))
    res = pl.pallas_call(
        body,
        out_shape=tuple(outs),
    )(*[a for a, _ in aggs], norm4, w0, b0.reshape(1, -1),
      w1, b1.reshape(1, -1))
    return res


def _tc_final(h3, wd, bd):
    def body(h_ref, wd_ref, bd_ref, o_ref):
        pooled = jnp.mean(h_ref[...], axis=0, keepdims=True)
        o_ref[...] = lax.dot_general(
            pooled, wd_ref[...], (((1,), (0,)), ((), ())),
            preferred_element_type=jnp.float32) + bd_ref[...]
    return pl.pallas_call(
        body,
        out_shape=jax.ShapeDtypeStruct((1, 1), jnp.float32),
    )(h3, wd, bd.reshape(1, 1))


def kernel(x, edge_index_r0, edge_index_r1,
           W1_r0, b1_r0, W1_r1, b1_r1,
           W2_r0, b2_r0, W2_r1, b2_r1,
           W3_r0, b3_r0, W3_r1, b3_r1,
           Wd, bd):
    src_flat = jnp.stack([edge_index_r0[0].reshape(_NT, _EC),
                          edge_index_r1[0].reshape(_NT, _EC)])
    dst_flat = jnp.stack([edge_index_r0[1].reshape(_NT, _EC),
                          edge_index_r1[1].reshape(_NT, _EC)])
    src_all = src_flat.reshape(2, _NT, _NCH, _CHK)
    dst_all = dst_flat.reshape(2, _NT, _NCH, _CHK)
    norm4 = _sc_norms(src_flat, dst_flat)

    hs = _tc_prescale(x, norm4)
    agg = _sc_segsum(src_all, dst_all, hs)
    h1, hs1 = _tc_layer([(agg, 64)], norm4,
                        W1_r0, b1_r0, W1_r1, b1_r1, 256, True)

    agg_lo = _sc_segsum(src_all, dst_all,
                        lax.slice_in_dim(hs1, 0, 128, axis=2))
    agg_hi = _sc_segsum(src_all, dst_all,
                        lax.slice_in_dim(hs1, 128, 256, axis=2))
    h2, hs2 = _tc_layer([(agg_lo, 128), (agg_hi, 128)], norm4,
                        W2_r0, b2_r0, W2_r1, b2_r1, 128, True)

    agg3 = _sc_segsum(src_all, dst_all, hs2)
    h3 = _tc_layer([(agg3, 128)], norm4,
                   W3_r0, b3_r0, W3_r1, b3_r1, 64, False)[0]

    return _tc_final(h3, Wd, bd)
